# baseline (speedup 1.0000x reference)
import jax
import jax.numpy as jnp
from jax import lax
from jax.experimental import pallas as pl
from jax.experimental.pallas import tpu as pltpu
from jax.experimental.pallas import tpu_sc as plsc

N = 10000
E = 320000
D = 128
H = 16
C = 47

NC = 2
NS = 16
NW = NC * NS
BLK = 128
NBUF = 4
LEAD = 2
NBLK1 = 160
NBLK2 = 80
NPAD = 10240
DUMMY = NPAD - 2
RPT = NPAD // NS
TPT = N // NS

_mesh = plsc.VectorSubcoreMesh(core_axis_name="c", subcore_axis_name="s")


def _ring_loop(table_sh, acc_sh, cnt_sh, src_idx, dst_idx, nblk,
               row_bufs, ones_buf, gsems, ssems, csems):
    for b in range(NBUF):
        pltpu.async_copy(table_sh.at[src_idx.at[b]], row_bufs[b], gsems[b])

    def group(g, carry):
        for b in range(NBUF):
            j = g * NBUF + b
            pltpu.make_async_copy(
                table_sh.at[src_idx.at[j]], row_bufs[b], gsems[b]).wait()
            pltpu.async_copy(
                row_bufs[b], acc_sh.at[dst_idx.at[j]], ssems[b], add=True)
            if cnt_sh is not None:
                pltpu.async_copy(
                    ones_buf, cnt_sh.at[dst_idx.at[j]], csems[b], add=True)

            bp = (b - LEAD) % NBUF

            @pl.when(jnp.logical_and(j >= LEAD, j + LEAD < nblk))
            def _():
                jp = j - LEAD
                pltpu.make_async_copy(
                    row_bufs[bp], acc_sh.at[dst_idx.at[jp]], ssems[bp]).wait()
                if cnt_sh is not None:
                    pltpu.make_async_copy(
                        ones_buf, cnt_sh.at[dst_idx.at[jp]], csems[bp]).wait()
                pltpu.async_copy(
                    table_sh.at[src_idx.at[j + LEAD]], row_bufs[bp], gsems[bp])
        return carry
    lax.fori_loop(0, nblk // NBUF, group, 0)

    for j in range(nblk - NBUF, nblk):
        b = j % NBUF
        pltpu.make_async_copy(
            row_bufs[b], acc_sh.at[dst_idx.at[j]], ssems[b]).wait()
        if cnt_sh is not None:
            pltpu.make_async_copy(
                ones_buf, cnt_sh.at[dst_idx.at[j]], csems[b]).wait()


def _zero_stage(stage):
    def zrow(i, carry):
        stage[i, :] = jnp.zeros((16,), jnp.float32)
        return carry
    lax.fori_loop(0, RPT, zrow, 0)


def _sage_sc_body(xp, xr, src1, dst1, src2, dst2,
                  acc2_out, cnt_out, h_out,
                  stage, xr_t, row_bufs, src_idx, dst_idx, ones_buf, zc,
                  table_sh, acc_sh, cnt_sh, sems):
    cid = lax.axis_index("c")
    sid = lax.axis_index("s")
    wid = sid * NC + cid
    gsems, ssems, csems = sems

    pltpu.sync_copy(xp.at[pl.ds(sid * RPT, RPT)], stage)
    pltpu.sync_copy(stage, table_sh.at[pl.ds(sid * RPT, RPT)])
    _zero_stage(stage)
    pltpu.sync_copy(stage, acc_sh.at[pl.ds(sid * RPT, RPT)])

    def zrow1(i, carry):
        zc[pl.ds(i * 16, 16)] = jnp.zeros((16,), jnp.float32)
        return carry
    lax.fori_loop(0, RPT // 16, zrow1, 0)
    pltpu.sync_copy(zc, cnt_sh.at[pl.ds(sid * RPT, RPT)])

    def orow(i, carry):
        ones_buf[pl.ds(i * 16, 16)] = jnp.ones((16,), jnp.float32)
        return carry
    lax.fori_loop(0, BLK // 16, orow, 0)
    plsc.subcore_barrier()

    pltpu.sync_copy(src1.at[sid], src_idx)
    pltpu.sync_copy(dst1.at[sid], dst_idx)
    _ring_loop(table_sh, acc_sh, cnt_sh, src_idx, dst_idx, NBLK1,
               row_bufs, ones_buf, gsems, ssems, csems)
    plsc.subcore_barrier()

    pltpu.sync_copy(acc_sh.at[pl.ds(sid * RPT, RPT)], stage)
    pltpu.sync_copy(cnt_sh.at[pl.ds(sid * RPT, RPT)], zc)
    pltpu.sync_copy(xr.at[pl.ds(sid * RPT, RPT)], xr_t)

    def hrow(g, carry):
        cv = zc[pl.ds(g * 16, 16)]
        for k in range(16):
            i = g * 16 + k
            c = jnp.maximum(cv[k], 1.0)
            stage[i, :] = jnp.maximum(stage[i, :] / c + xr_t[i, :], 0.0)
        return carry
    lax.fori_loop(0, RPT // 16, hrow, 0)
    pltpu.sync_copy(stage, table_sh.at[pl.ds(sid * RPT, RPT)])

    @pl.when(cid == 0)
    def _():
        pltpu.sync_copy(stage, h_out.at[pl.ds(sid * RPT, RPT)])

    pltpu.sync_copy(cnt_sh.at[pl.ds(sid * RPT, RPT)], zc)
    pltpu.sync_copy(zc, cnt_out.at[cid, pl.ds(sid * RPT, RPT)])
    _zero_stage(stage)
    pltpu.sync_copy(stage, acc_sh.at[pl.ds(sid * RPT, RPT)])
    plsc.subcore_barrier()

    pltpu.sync_copy(src2.at[wid], src_idx.at[pl.ds(0, NBLK2)])
    pltpu.sync_copy(dst2.at[wid], dst_idx.at[pl.ds(0, NBLK2)])
    _ring_loop(table_sh, acc_sh, None, src_idx, dst_idx, NBLK2,
               row_bufs, ones_buf, gsems, ssems, csems)
    plsc.subcore_barrier()

    pltpu.sync_copy(acc_sh.at[pl.ds(sid * RPT, RPT)], stage)
    pltpu.sync_copy(stage, acc2_out.at[cid, pl.ds(sid * RPT, RPT)])


_sage_sc = pl.kernel(
    _sage_sc_body,
    out_type=(
        jax.ShapeDtypeStruct((NC, NPAD, H), jnp.float32),
        jax.ShapeDtypeStruct((NC, NPAD), jnp.float32),
        jax.ShapeDtypeStruct((NPAD, H), jnp.float32),
    ),
    mesh=_mesh,
    scratch_types=(
        pltpu.VMEM((RPT, H), jnp.float32),
        pltpu.VMEM((RPT, H), jnp.float32),
        [pltpu.VMEM((BLK, H), jnp.float32) for _ in range(NBUF)],
        pltpu.VMEM((NBLK1, BLK), jnp.int32),
        pltpu.VMEM((NBLK1, BLK), jnp.int32),
        pltpu.VMEM((BLK,), jnp.float32),
        pltpu.VMEM((RPT,), jnp.float32),
        pltpu.VMEM_SHARED((NPAD, H), jnp.float32),
        pltpu.VMEM_SHARED((NPAD, H), jnp.float32),
        pltpu.VMEM_SHARED((NPAD,), jnp.float32),
        [[pltpu.SemaphoreType.DMA for _ in range(NBUF)] for _ in range(3)],
    ),
    compiler_params=pltpu.CompilerParams(use_tc_tiling_on_sc=False),
)

_ROWS = 400
_GRID = N // _ROWS


def _proj_body(x_ref, wl_ref, wr_ref, b_ref, xp_ref, xr_ref):
    xb = x_ref[...]
    dn = (((1,), (1,)), ((), ()))
    xp_ref[...] = lax.dot_general(xb, wl_ref[...], dn,
                                  preferred_element_type=jnp.float32)
    xr_ref[...] = lax.dot_general(xb, wr_ref[...], dn,
                                  preferred_element_type=jnp.float32) + b_ref[...]


def _out_body(acc_ref, cnt_ref, h_ref, wl_ref, wr_ref, b_ref, o_ref):
    a = acc_ref[0] + acc_ref[1]
    c = jnp.maximum(cnt_ref[...], 1.0)
    m = a / c
    dn = (((1,), (1,)), ((), ()))
    o_ref[...] = (lax.dot_general(m, wl_ref[...], dn,
                                  preferred_element_type=jnp.float32)
                  + lax.dot_general(h_ref[...], wr_ref[...], dn,
                                    preferred_element_type=jnp.float32)
                  + b_ref[...])


def kernel(x, edge_index, W1l, b1, W1r, W2l, b2, W2r):
    def _pad_edges(e, nchunk, nblk, fill_dst):
        per = nchunk * nblk * BLK // nchunk
        padw = per - E // nchunk
        if fill_dst:
            fill = N + (jnp.arange(padw, dtype=jnp.int32) % (NPAD - N))
        else:
            fill = jnp.arange(padw, dtype=jnp.int32) % 128
        return jnp.concatenate(
            [e.reshape(nchunk, E // nchunk),
             jnp.broadcast_to(fill, (nchunk, padw))], axis=1
        ).reshape(nchunk, nblk, BLK)

    src1 = _pad_edges(edge_index[0], NS, NBLK1, False)
    dst1 = _pad_edges(edge_index[1], NS, NBLK1, True)
    src2 = _pad_edges(edge_index[0], NW, NBLK2, False)
    dst2 = _pad_edges(edge_index[1], NW, NBLK2, True)

    xp, xr = pl.pallas_call(
        _proj_body,
        grid=(_GRID,),
        in_specs=[
            pl.BlockSpec((_ROWS, D), lambda i: (i, 0)),
            pl.BlockSpec((H, D), lambda i: (0, 0)),
            pl.BlockSpec((H, D), lambda i: (0, 0)),
            pl.BlockSpec((1, H), lambda i: (0, 0)),
        ],
        out_specs=[
            pl.BlockSpec((_ROWS, H), lambda i: (i, 0)),
            pl.BlockSpec((_ROWS, H), lambda i: (i, 0)),
        ],
        out_shape=[
            jax.ShapeDtypeStruct((NPAD, H), jnp.float32),
            jax.ShapeDtypeStruct((NPAD, H), jnp.float32),
        ],
    )(x, W1l, W1r, b1[None, :])

    acc2, cnt, h = _sage_sc(xp, xr, src1, dst1, src2, dst2)
    cnt0 = cnt[0][:, None]

    out = pl.pallas_call(
        _out_body,
        grid=(_GRID,),
        in_specs=[
            pl.BlockSpec((NC, _ROWS, H), lambda i: (0, i, 0)),
            pl.BlockSpec((_ROWS, 1), lambda i: (i, 0)),
            pl.BlockSpec((_ROWS, H), lambda i: (i, 0)),
            pl.BlockSpec((C, H), lambda i: (0, 0)),
            pl.BlockSpec((C, H), lambda i: (0, 0)),
            pl.BlockSpec((1, C), lambda i: (0, 0)),
        ],
        out_specs=pl.BlockSpec((_ROWS, C), lambda i: (i, 0)),
        out_shape=jax.ShapeDtypeStruct((N, C), jnp.float32),
    )(acc2, cnt0, h, W2l, W2r, b2[None, :])

    return out


# speedup vs baseline: 1.1120x; 1.1120x over previous
import jax
import jax.numpy as jnp
from jax import lax
from jax.experimental import pallas as pl
from jax.experimental.pallas import tpu as pltpu
from jax.experimental.pallas import tpu_sc as plsc

N = 10000
E = 320000
D = 128
H = 16
C = 47

NC = 2
NS = 16
NW = NC * NS
BLK = 128
NBUF = 8
LEAD = 4
NBLK1 = 160
NBLK2 = 80
NPAD = 10240
DUMMY = NPAD - 2
RPT = NPAD // NS
TPT = N // NS

_mesh = plsc.VectorSubcoreMesh(core_axis_name="c", subcore_axis_name="s")


def _ring_loop(table_sh, acc_sh, cnt_sh, src_idx, dst_idx, nblk,
               row_bufs, ones_buf, gsems, ssems, csems):
    for b in range(NBUF):
        pltpu.async_copy(table_sh.at[src_idx.at[b]], row_bufs[b], gsems[b])

    def group(g, carry):
        for b in range(NBUF):
            j = g * NBUF + b
            pltpu.make_async_copy(
                table_sh.at[src_idx.at[j]], row_bufs[b], gsems[b]).wait()
            pltpu.async_copy(
                row_bufs[b], acc_sh.at[dst_idx.at[j]], ssems[b], add=True)
            if cnt_sh is not None:
                pltpu.async_copy(
                    ones_buf, cnt_sh.at[dst_idx.at[j]], csems[b], add=True)

            bp = (b - LEAD) % NBUF

            @pl.when(jnp.logical_and(j >= LEAD, j + LEAD < nblk))
            def _():
                jp = j - LEAD
                pltpu.make_async_copy(
                    row_bufs[bp], acc_sh.at[dst_idx.at[jp]], ssems[bp]).wait()
                if cnt_sh is not None:
                    pltpu.make_async_copy(
                        ones_buf, cnt_sh.at[dst_idx.at[jp]], csems[bp]).wait()
                pltpu.async_copy(
                    table_sh.at[src_idx.at[j + LEAD]], row_bufs[bp], gsems[bp])
        return carry
    lax.fori_loop(0, nblk // NBUF, group, 0)

    for j in range(nblk - NBUF, nblk):
        b = j % NBUF
        pltpu.make_async_copy(
            row_bufs[b], acc_sh.at[dst_idx.at[j]], ssems[b]).wait()
        if cnt_sh is not None:
            pltpu.make_async_copy(
                ones_buf, cnt_sh.at[dst_idx.at[j]], csems[b]).wait()


def _zero_stage(stage):
    def zrow(i, carry):
        stage[i, :] = jnp.zeros((16,), jnp.float32)
        return carry
    lax.fori_loop(0, RPT, zrow, 0)


def _sage_sc_body(xp, xr, src1, dst1, src2, dst2,
                  acc2_out, cnt_out, h_out,
                  stage, xr_t, zbuf, row_bufs, src_idx, dst_idx,
                  src_idx2, dst_idx2, ones_buf, zc,
                  table_sh, acc_sh, cnt_sh, sems):
    cid = lax.axis_index("c")
    sid = lax.axis_index("s")
    wid = sid * NC + cid
    gsems, ssems, csems, esems = sems

    pltpu.async_copy(xr.at[pl.ds(sid * RPT, RPT)], xr_t, esems[4])
    pltpu.async_copy(src2.at[wid], src_idx2, esems[2])
    pltpu.async_copy(dst2.at[wid], dst_idx2, esems[3])
    pltpu.sync_copy(xp.at[pl.ds(sid * RPT, RPT)], stage)
    pltpu.sync_copy(stage, table_sh.at[pl.ds(sid * RPT, RPT)])
    _zero_stage(zbuf)
    pltpu.sync_copy(zbuf, acc_sh.at[pl.ds(sid * RPT, RPT)])

    def zrow1(i, carry):
        zc[pl.ds(i * 16, 16)] = jnp.zeros((16,), jnp.float32)
        return carry
    lax.fori_loop(0, RPT // 16, zrow1, 0)
    pltpu.sync_copy(zc, cnt_sh.at[pl.ds(sid * RPT, RPT)])

    def orow(i, carry):
        ones_buf[pl.ds(i * 16, 16)] = jnp.ones((16,), jnp.float32)
        return carry
    lax.fori_loop(0, BLK // 16, orow, 0)
    plsc.subcore_barrier()

    pltpu.sync_copy(src1.at[sid], src_idx)
    pltpu.sync_copy(dst1.at[sid], dst_idx)
    _ring_loop(table_sh, acc_sh, cnt_sh, src_idx, dst_idx, NBLK1,
               row_bufs, ones_buf, gsems, ssems, csems)
    plsc.subcore_barrier()

    pltpu.sync_copy(acc_sh.at[pl.ds(sid * RPT, RPT)], stage)
    pltpu.sync_copy(cnt_sh.at[pl.ds(sid * RPT, RPT)], zc)
    pltpu.make_async_copy(xr.at[pl.ds(sid * RPT, RPT)], xr_t, esems[4]).wait()

    def hrow(g, carry):
        cv = zc[pl.ds(g * 16, 16)]
        for k in range(16):
            i = g * 16 + k
            c = jnp.maximum(cv[k], 1.0)
            stage[i, :] = jnp.maximum(stage[i, :] / c + xr_t[i, :], 0.0)
        return carry
    lax.fori_loop(0, RPT // 16, hrow, 0)
    pltpu.sync_copy(stage, table_sh.at[pl.ds(sid * RPT, RPT)])

    @pl.when(cid == 0)
    def _():
        pltpu.async_copy(stage, h_out.at[pl.ds(sid * RPT, RPT)], esems[0])

    pltpu.async_copy(zc, cnt_out.at[cid, pl.ds(sid * RPT, RPT)], esems[1])
    pltpu.sync_copy(zbuf, acc_sh.at[pl.ds(sid * RPT, RPT)])
    plsc.subcore_barrier()

    pltpu.make_async_copy(src2.at[wid], src_idx2, esems[2]).wait()
    pltpu.make_async_copy(dst2.at[wid], dst_idx2, esems[3]).wait()
    _ring_loop(table_sh, acc_sh, None, src_idx2, dst_idx2, NBLK2,
               row_bufs, ones_buf, gsems, ssems, csems)
    plsc.subcore_barrier()

    @pl.when(cid == 0)
    def _():
        pltpu.make_async_copy(stage, h_out.at[pl.ds(sid * RPT, RPT)],
                              esems[0]).wait()

    pltpu.make_async_copy(zc, cnt_out.at[cid, pl.ds(sid * RPT, RPT)],
                          esems[1]).wait()
    pltpu.sync_copy(acc_sh.at[pl.ds(sid * RPT, RPT)], stage)
    pltpu.sync_copy(stage, acc2_out.at[cid, pl.ds(sid * RPT, RPT)])


_sage_sc = pl.kernel(
    _sage_sc_body,
    out_type=(
        jax.ShapeDtypeStruct((NC, NPAD, H), jnp.float32),
        jax.ShapeDtypeStruct((NC, NPAD), jnp.float32),
        jax.ShapeDtypeStruct((NPAD, H), jnp.float32),
    ),
    mesh=_mesh,
    scratch_types=(
        pltpu.VMEM((RPT, H), jnp.float32),
        pltpu.VMEM((RPT, H), jnp.float32),
        pltpu.VMEM((RPT, H), jnp.float32),
        [pltpu.VMEM((BLK, H), jnp.float32) for _ in range(NBUF)],
        pltpu.VMEM((NBLK1, BLK), jnp.int32),
        pltpu.VMEM((NBLK1, BLK), jnp.int32),
        pltpu.VMEM((NBLK2, BLK), jnp.int32),
        pltpu.VMEM((NBLK2, BLK), jnp.int32),
        pltpu.VMEM((BLK,), jnp.float32),
        pltpu.VMEM((RPT,), jnp.float32),
        pltpu.VMEM_SHARED((NPAD, H), jnp.float32),
        pltpu.VMEM_SHARED((NPAD, H), jnp.float32),
        pltpu.VMEM_SHARED((NPAD,), jnp.float32),
        [[pltpu.SemaphoreType.DMA for _ in range(NBUF)] for _ in range(3)]
        + [[pltpu.SemaphoreType.DMA for _ in range(5)]],
    ),
    compiler_params=pltpu.CompilerParams(use_tc_tiling_on_sc=False),
)

_ROWS = 400
_GRID = N // _ROWS


def _proj_body(x_ref, wl_ref, wr_ref, b_ref, xp_ref, xr_ref):
    xb = x_ref[...]
    dn = (((1,), (1,)), ((), ()))
    xp_ref[...] = lax.dot_general(xb, wl_ref[...], dn,
                                  preferred_element_type=jnp.float32)
    xr_ref[...] = lax.dot_general(xb, wr_ref[...], dn,
                                  preferred_element_type=jnp.float32) + b_ref[...]


def _out_body(acc_ref, cnt_ref, h_ref, wl_ref, wr_ref, b_ref, o_ref):
    a = acc_ref[0] + acc_ref[1]
    c = jnp.maximum(cnt_ref[...], 1.0)
    m = a / c
    dn = (((1,), (1,)), ((), ()))
    o_ref[...] = (lax.dot_general(m, wl_ref[...], dn,
                                  preferred_element_type=jnp.float32)
                  + lax.dot_general(h_ref[...], wr_ref[...], dn,
                                    preferred_element_type=jnp.float32)
                  + b_ref[...])


def kernel(x, edge_index, W1l, b1, W1r, W2l, b2, W2r):
    def _pad_edges(e, nchunk, nblk, fill_dst):
        per = nchunk * nblk * BLK // nchunk
        padw = per - E // nchunk
        if fill_dst:
            fill = N + (jnp.arange(padw, dtype=jnp.int32) % (NPAD - N))
        else:
            fill = jnp.arange(padw, dtype=jnp.int32) % 128
        return jnp.concatenate(
            [e.reshape(nchunk, E // nchunk),
             jnp.broadcast_to(fill, (nchunk, padw))], axis=1
        ).reshape(nchunk, nblk, BLK)

    src1 = _pad_edges(edge_index[0], NS, NBLK1, False)
    dst1 = _pad_edges(edge_index[1], NS, NBLK1, True)
    src2 = _pad_edges(edge_index[0], NW, NBLK2, False)
    dst2 = _pad_edges(edge_index[1], NW, NBLK2, True)

    xp, xr = pl.pallas_call(
        _proj_body,
        grid=(_GRID,),
        in_specs=[
            pl.BlockSpec((_ROWS, D), lambda i: (i, 0)),
            pl.BlockSpec((H, D), lambda i: (0, 0)),
            pl.BlockSpec((H, D), lambda i: (0, 0)),
            pl.BlockSpec((1, H), lambda i: (0, 0)),
        ],
        out_specs=[
            pl.BlockSpec((_ROWS, H), lambda i: (i, 0)),
            pl.BlockSpec((_ROWS, H), lambda i: (i, 0)),
        ],
        out_shape=[
            jax.ShapeDtypeStruct((NPAD, H), jnp.float32),
            jax.ShapeDtypeStruct((NPAD, H), jnp.float32),
        ],
    )(x, W1l, W1r, b1[None, :])

    acc2, cnt, h = _sage_sc(xp, xr, src1, dst1, src2, dst2)
    cnt0 = cnt[0][:, None]

    out = pl.pallas_call(
        _out_body,
        grid=(_GRID,),
        in_specs=[
            pl.BlockSpec((NC, _ROWS, H), lambda i: (0, i, 0)),
            pl.BlockSpec((_ROWS, 1), lambda i: (i, 0)),
            pl.BlockSpec((_ROWS, H), lambda i: (i, 0)),
            pl.BlockSpec((C, H), lambda i: (0, 0)),
            pl.BlockSpec((C, H), lambda i: (0, 0)),
            pl.BlockSpec((1, C), lambda i: (0, 0)),
        ],
        out_specs=pl.BlockSpec((_ROWS, C), lambda i: (i, 0)),
        out_shape=jax.ShapeDtypeStruct((N, C), jnp.float32),
    )(acc2, cnt0, h, W2l, W2r, b2[None, :])

    return out
